# baseline (speedup 1.0000x reference)
import functools

import jax
import jax.numpy as jnp
from jax import lax
from jax.experimental import pallas as pl
from jax.experimental.pallas import tpu as pltpu
from jax.experimental.pallas import tpu_sc as plsc

_MESH = plsc.VectorSubcoreMesh(
    core_axis_name="c", subcore_axis_name="s", num_cores=1, num_subcores=1
)

def _body(x_hbm, w_hbm, b_hbm, out_hbm, xv, wv, bv, fv, ov, sem):
    wid = lax.axis_index("s") * 2 + lax.axis_index("c")

    @pl.when(wid == 0)
    def _():
        cx = pltpu.make_async_copy(x_hbm, xv.at[pl.ds(0, 8)], sem)
        cw = pltpu.make_async_copy(w_hbm, wv.at[pl.ds(0, 24)], sem)
        cb = pltpu.make_async_copy(b_hbm, bv.at[pl.ds(0, 4)], sem)
        cx.start()
        cw.start()
        cb.start()
        cx.wait()
        cw.wait()
        cb.wait()

        lane = lax.iota(jnp.int32, 16)
        zi = jnp.zeros((16,), jnp.int32)
        zf = jnp.zeros((16,), jnp.float32)
        lt2, lt4, lt6 = lane < 2, lane < 4, lane < 6
        i1 = jnp.where(lt2, lane + 4, jnp.where(lt4, lane - 2, jnp.where(lt6, lane + 2, zi)))
        c1 = jnp.where(lt4, 1.0, jnp.where(lt6, 3.0, zf))
        i2 = jnp.where(lt2, lane + 6, jnp.where(lt4, zi, jnp.where(lt6, lane + 2, zi)))
        c2 = jnp.where(lt2, 2.0, zf)
        flat = plsc.load_gather(xv, [i1]) * c1 + plsc.load_gather(xv, [i2]) * c2
        fv[pl.ds(8, 16)] = flat

        lane4 = jnp.where(lt4, lane, zi)
        acc = plsc.load_gather(bv, [lane4])
        for k in range(6):
            col = plsc.load_gather(wv, [jnp.where(lt4, lane * 6 + k, zi)])
            fk = plsc.load_gather(fv, [zi + (8 + k)])
            acc = acc + col * fk
        ov[...] = acc
        pltpu.sync_copy(ov.at[pl.ds(0, 4)], out_hbm)


@jax.jit
def _run(xf, wf, b):
    k = functools.partial(
        pl.kernel,
        out_type=jax.ShapeDtypeStruct((4,), jnp.float32),
        mesh=_MESH,
        scratch_types=[
            pltpu.VMEM((16,), jnp.float32),
            pltpu.VMEM((24,), jnp.float32),
            pltpu.VMEM((16,), jnp.float32),
            pltpu.VMEM((24,), jnp.float32),
            pltpu.VMEM((16,), jnp.float32),
            pltpu.SemaphoreType.DMA,
        ],
        compiler_params=pltpu.CompilerParams(needs_layout_passes=False),
    )(_body)
    return k(xf, wf, b)


def kernel(x, W, b):
    return _run(x.reshape(8), W.reshape(24), b)


# speedup vs baseline: 1.0784x; 1.0784x over previous
import functools

import jax
import jax.numpy as jnp
from jax.experimental import pallas as pl
from jax.experimental.pallas import tpu as pltpu
from jax.experimental.pallas import tpu_sc as plsc

_MESH = plsc.ScalarSubcoreMesh(axis_name="c", num_cores=1)


def _body(x_hbm, w_hbm, b_hbm, out_hbm, xv, wv, bv, ov, sem):
    cx = pltpu.make_async_copy(x_hbm, xv, sem)
    cw = pltpu.make_async_copy(w_hbm, wv, sem)
    cb = pltpu.make_async_copy(b_hbm, bv, sem)
    cx.start()
    cw.start()
    cb.start()
    cx.wait()
    cw.wait()
    cb.wait()

    xs = [xv[i] for i in range(8)]
    flat = (
        xs[4] + 2.0 * xs[6],
        xs[5] + 2.0 * xs[7],
        xs[0],
        xs[1],
        3.0 * xs[6],
        3.0 * xs[7],
    )
    for j in range(4):
        acc = bv[j]
        for k in range(6):
            acc = acc + flat[k] * wv[6 * j + k]
        ov[j] = acc
    pltpu.sync_copy(ov, out_hbm)


@jax.jit
def _run(xf, wf, b):
    k = functools.partial(
        pl.kernel,
        out_type=jax.ShapeDtypeStruct((4,), jnp.float32),
        mesh=_MESH,
        scratch_types=[
            pltpu.SMEM((8,), jnp.float32),
            pltpu.SMEM((24,), jnp.float32),
            pltpu.SMEM((4,), jnp.float32),
            pltpu.SMEM((4,), jnp.float32),
            pltpu.SemaphoreType.DMA,
        ],
        compiler_params=pltpu.CompilerParams(needs_layout_passes=False),
    )(_body)
    return k(xf, wf, b)


def kernel(x, W, b):
    return _run(x.reshape(8), W.reshape(24), b)


# speedup vs baseline: 1.0900x; 1.0107x over previous
import functools

import jax
import jax.numpy as jnp
from jax.experimental import pallas as pl
from jax.experimental.pallas import tpu as pltpu
from jax.experimental.pallas import tpu_sc as plsc

_MESH = plsc.ScalarSubcoreMesh(axis_name="c", num_cores=1)


def _body(x_hbm, w_hbm, b_hbm, out_hbm, xv, wv, bv, ov, sem):
    cx = pltpu.make_async_copy(x_hbm, xv, sem)
    cw = pltpu.make_async_copy(w_hbm, wv, sem)
    cb = pltpu.make_async_copy(b_hbm, bv, sem)
    cx.start()
    cw.start()
    cb.start()
    cx.wait()

    xs = [xv[i] for i in range(8)]
    flat = (
        xs[4] + 2.0 * xs[6],
        xs[5] + 2.0 * xs[7],
        xs[0],
        xs[1],
        3.0 * xs[6],
        3.0 * xs[7],
    )
    cw.wait()
    cb.wait()
    for j in range(4):
        acc = bv[j]
        for k in range(6):
            acc = acc + flat[k] * wv[6 * j + k]
        ov[j] = acc
    pltpu.sync_copy(ov, out_hbm)


@jax.jit
def _run(xf, wf, b):
    k = functools.partial(
        pl.kernel,
        out_type=jax.ShapeDtypeStruct((4,), jnp.float32),
        mesh=_MESH,
        scratch_types=[
            pltpu.SMEM((8,), jnp.float32),
            pltpu.SMEM((24,), jnp.float32),
            pltpu.SMEM((4,), jnp.float32),
            pltpu.SMEM((4,), jnp.float32),
            pltpu.SemaphoreType.DMA,
        ],
        compiler_params=pltpu.CompilerParams(needs_layout_passes=False),
    )(_body)
    return k(xf, wf, b)


def kernel(x, W, b):
    return _run(x.reshape(8), W.reshape(24), b)
